# baseline (speedup 1.0000x reference)
import dataclasses
import functools

import jax
import jax.numpy as jnp
from jax import lax
from jax.experimental import pallas as pl
from jax.experimental.pallas import tpu as pltpu
from jax.experimental.pallas import tpu_sc as plsc


def _router_body(T, E, n_total, x_ref, w1t_ref, b1_ref, w2_ref, b2_ref,
                 i0_ref, i1_ref, g0_ref, g1_ref):
    i = pl.program_id(0)
    h = jnp.dot(x_ref[...], w1t_ref[...], preferred_element_type=jnp.float32)
    h = jnp.maximum(h + b1_ref[...], 0.0)
    logits = lax.dot_general(w2_ref[...], h, (((1,), (1,)), ((), ())),
                             preferred_element_type=jnp.float32)
    logits = logits + b2_ref[...]
    m = jnp.max(logits, axis=0, keepdims=True)
    p = jnp.exp(logits - m)
    p = p / jnp.sum(p, axis=0, keepdims=True)
    sub = lax.broadcasted_iota(jnp.int32, (E, T), 0)
    p1 = jnp.max(p, axis=0, keepdims=True)
    e1 = jnp.min(jnp.where(p == p1, sub, E), axis=0, keepdims=True)
    pm = jnp.where(sub == e1, -jnp.inf, p)
    p2 = jnp.max(pm, axis=0, keepdims=True)
    e2 = jnp.min(jnp.where(pm == p2, sub, E), axis=0, keepdims=True)
    tok = i * T + lax.broadcasted_iota(jnp.int32, (1, T), 1)
    i0_ref[0] = e1 * n_total + tok
    i1_ref[0] = e2 * n_total + tok
    g0_ref[0] = p1
    g1_ref[0] = p2


def _router(x, w1t, b1, w2, b2, T=1024):
    N, H = x.shape
    E = w2.shape[0]
    nb = N // T
    body = functools.partial(_router_body, T, E, N)
    outs = pl.pallas_call(
        body,
        grid=(nb,),
        in_specs=[
            pl.BlockSpec((T, H), lambda i: (i, 0)),
            pl.BlockSpec((H, H), lambda i: (0, 0)),
            pl.BlockSpec((1, H), lambda i: (0, 0)),
            pl.BlockSpec((E, H), lambda i: (0, 0)),
            pl.BlockSpec((E, 1), lambda i: (0, 0)),
        ],
        out_specs=[
            pl.BlockSpec((1, 1, T), lambda i: (i, 0, 0)),
            pl.BlockSpec((1, 1, T), lambda i: (i, 0, 0)),
            pl.BlockSpec((1, 1, T), lambda i: (i, 0, 0)),
            pl.BlockSpec((1, 1, T), lambda i: (i, 0, 0)),
        ],
        out_shape=[
            jax.ShapeDtypeStruct((nb, 1, T), jnp.int32),
            jax.ShapeDtypeStruct((nb, 1, T), jnp.int32),
            jax.ShapeDtypeStruct((nb, 1, T), jnp.float32),
            jax.ShapeDtypeStruct((nb, 1, T), jnp.float32),
        ],
    )(x, w1t, b1, w2, b2)
    return outs


def _make_combine(N, H, G=8):
    n_workers = 32
    per_w = N // n_workers
    n_chunks = per_w // G
    assert n_chunks % 4 == 0
    mesh = plsc.VectorSubcoreMesh(
        core_axis_name="c", subcore_axis_name="s", num_cores=2, num_subcores=16)

    cp = pltpu.CompilerParams()
    if "needs_layout_passes" in pltpu.CompilerParams.__dataclass_fields__:
        cp = dataclasses.replace(cp, needs_layout_passes=False)

    @functools.partial(
        pl.kernel,
        out_type=jax.ShapeDtypeStruct((N, H), jnp.float32),
        mesh=mesh,
        compiler_params=cp,
        scratch_types=[
            pltpu.VMEM((2 * per_w,), jnp.int32),
            pltpu.VMEM((2 * per_w,), jnp.float32),
            pltpu.VMEM((4, 2 * G, H), jnp.float32),
            pltpu.VMEM((4, G, H), jnp.float32),
            pltpu.SemaphoreType.DMA,
            pltpu.SemaphoreType.DMA,
            pltpu.SemaphoreType.DMA,
            pltpu.SemaphoreType.DMA,
            pltpu.SemaphoreType.DMA,
            pltpu.SemaphoreType.DMA,
            pltpu.SemaphoreType.DMA,
            pltpu.SemaphoreType.DMA,
        ],
    )
    def combine(table, i0, i1, g0, g1, out,
                idx_v, g_v, r_v, o_v,
                sg0, sg1, sg2, sg3, so0, so1, so2, so3):
        wid = lax.axis_index("s") * 2 + lax.axis_index("c")
        wbase = wid * per_w
        sg = (sg0, sg1, sg2, sg3)
        so = (so0, so1, so2, so3)

        pre_descs = (
            pltpu.make_async_copy(
                i0.at[pl.ds(wbase, per_w)], idx_v.at[pl.ds(0, per_w)], sg0),
            pltpu.make_async_copy(
                i1.at[pl.ds(wbase, per_w)], idx_v.at[pl.ds(per_w, per_w)],
                sg0),
            pltpu.make_async_copy(
                g0.at[pl.ds(wbase, per_w)], g_v.at[pl.ds(0, per_w)], sg0),
            pltpu.make_async_copy(
                g1.at[pl.ds(wbase, per_w)], g_v.at[pl.ds(per_w, per_w)], sg0),
        )
        for d in pre_descs:
            d.start()
        for d in pre_descs:
            d.wait()

        def gather_descs(cc, b):
            base = cc * G
            d0 = pltpu.make_async_copy(
                table.at[idx_v.at[pl.ds(base, G)]],
                r_v.at[b, pl.ds(0, G)], sg[b])
            d1 = pltpu.make_async_copy(
                table.at[idx_v.at[pl.ds(per_w + base, G)]],
                r_v.at[b, pl.ds(G, G)], sg[b])
            return (d0, d1)

        def out_desc(cc, b):
            return pltpu.make_async_copy(
                o_v.at[b], out.at[pl.ds(wbase + cc * G, G)], so[b])

        for pre in range(3):
            for d in gather_descs(pre, pre):
                d.start()

        @pl.loop(0, n_chunks, step=4)
        def _quad(c):
            for b in range(4):
                cc = c + b
                for d in gather_descs(cc, b):
                    d.wait()

                @pl.when(cc + 3 < n_chunks)
                def _():
                    for d in gather_descs(cc + 3, (b + 3) % 4):
                        d.start()

                @pl.when(cc >= 4)
                def _():
                    out_desc(cc, b).wait()

                for t in range(G):
                    ia = jnp.full((16,), cc * G + t, jnp.int32)
                    ga = plsc.load_gather(g_v, [ia])
                    gb = plsc.load_gather(g_v, [ia + per_w])
                    for j in range(H // 16):
                        sl = pl.ds(j * 16, 16)
                        o_v[b, t, sl] = (r_v[b, t, sl] * ga
                                         + r_v[b, G + t, sl] * gb)

                out_desc(cc, b).start()

        for tail in range(4):
            out_desc(n_chunks - 4 + tail, tail).wait()

    return combine


def kernel(hidden_states, expert_outputs, W1, b1, W2, b2):
    B, S, H = hidden_states.shape
    E = W2.shape[0]
    N = B * S
    x = hidden_states.reshape(N, H)
    table = expert_outputs.reshape(E * N, H)
    i0r, i1r, g0r, g1r = _router(
        x, W1.T, b1.reshape(1, H), W2, b2.reshape(E, 1))
    i0 = i0r.reshape(N)
    i1 = i1r.reshape(N)
    g0 = g0r.reshape(N)
    g1 = g1r.reshape(N)
    out = _make_combine(N, H)(table, i0, i1, g0, g1)
    return out.reshape(B, S, H)


# speedup vs baseline: 1.3796x; 1.3796x over previous
import dataclasses
import functools

import jax
import jax.numpy as jnp
from jax import lax
from jax.experimental import pallas as pl
from jax.experimental.pallas import tpu as pltpu
from jax.experimental.pallas import tpu_sc as plsc


def _router_body(T, E, n_total, x_ref, w1t_ref, b1_ref, w2_ref, b2_ref,
                 i0_ref, i1_ref, g0_ref, g1_ref):
    i = pl.program_id(0)
    h = jnp.dot(x_ref[...], w1t_ref[...], preferred_element_type=jnp.float32)
    h = jnp.maximum(h + b1_ref[...], 0.0)
    logits = lax.dot_general(w2_ref[...], h, (((1,), (1,)), ((), ())),
                             preferred_element_type=jnp.float32)
    logits = logits + b2_ref[...]
    m = jnp.max(logits, axis=0, keepdims=True)
    p = jnp.exp(logits - m)
    p = p / jnp.sum(p, axis=0, keepdims=True)
    sub = lax.broadcasted_iota(jnp.int32, (E, T), 0)
    p1 = jnp.max(p, axis=0, keepdims=True)
    e1 = jnp.min(jnp.where(p == p1, sub, E), axis=0, keepdims=True)
    pm = jnp.where(sub == e1, -jnp.inf, p)
    p2 = jnp.max(pm, axis=0, keepdims=True)
    e2 = jnp.min(jnp.where(pm == p2, sub, E), axis=0, keepdims=True)
    tok = i * T + lax.broadcasted_iota(jnp.int32, (1, T), 1)
    i0_ref[0] = e1 * n_total + tok
    i1_ref[0] = e2 * n_total + tok
    g0_ref[0] = p1
    g1_ref[0] = p2


def _router(x, w1t, b1, w2, b2, T=1024):
    N, H = x.shape
    E = w2.shape[0]
    nb = N // T
    body = functools.partial(_router_body, T, E, N)
    outs = pl.pallas_call(
        body,
        grid=(nb,),
        in_specs=[
            pl.BlockSpec((T, H), lambda i: (i, 0)),
            pl.BlockSpec((H, H), lambda i: (0, 0)),
            pl.BlockSpec((1, H), lambda i: (0, 0)),
            pl.BlockSpec((E, H), lambda i: (0, 0)),
            pl.BlockSpec((E, 1), lambda i: (0, 0)),
        ],
        out_specs=[
            pl.BlockSpec((1, 1, T), lambda i: (i, 0, 0)),
            pl.BlockSpec((1, 1, T), lambda i: (i, 0, 0)),
            pl.BlockSpec((1, 1, T), lambda i: (i, 0, 0)),
            pl.BlockSpec((1, 1, T), lambda i: (i, 0, 0)),
        ],
        out_shape=[
            jax.ShapeDtypeStruct((nb, 1, T), jnp.int32),
            jax.ShapeDtypeStruct((nb, 1, T), jnp.int32),
            jax.ShapeDtypeStruct((nb, 1, T), jnp.float32),
            jax.ShapeDtypeStruct((nb, 1, T), jnp.float32),
        ],
    )(x, w1t, b1, w2, b2)
    return outs


def _make_combine(N, H, G=8):
    n_workers = 32
    per_w = N // n_workers
    n_chunks = per_w // G
    assert n_chunks % 4 == 0
    mesh = plsc.VectorSubcoreMesh(
        core_axis_name="c", subcore_axis_name="s", num_cores=2, num_subcores=16)

    cp = pltpu.CompilerParams()
    if "needs_layout_passes" in pltpu.CompilerParams.__dataclass_fields__:
        cp = dataclasses.replace(cp, needs_layout_passes=False)

    @functools.partial(
        pl.kernel,
        out_type=jax.ShapeDtypeStruct((N, H), jnp.float32),
        mesh=mesh,
        compiler_params=cp,
        scratch_types=[
            pltpu.VMEM((2 * per_w,), jnp.int32),
            pltpu.VMEM((2 * per_w,), jnp.float32),
            pltpu.VMEM((4, 2 * G, H), jnp.float32),
            pltpu.VMEM((4, G, H), jnp.float32),
            pltpu.SemaphoreType.DMA,
            pltpu.SemaphoreType.DMA,
            pltpu.SemaphoreType.DMA,
            pltpu.SemaphoreType.DMA,
            pltpu.SemaphoreType.DMA,
            pltpu.SemaphoreType.DMA,
            pltpu.SemaphoreType.DMA,
            pltpu.SemaphoreType.DMA,
        ],
    )
    def combine(table, i0, i1, g0, g1, out,
                idx_v, g_v, r_v, o_v,
                sg0, sg1, sg2, sg3, so0, so1, so2, so3):
        wid = lax.axis_index("s") * 2 + lax.axis_index("c")
        wbase = wid * per_w
        sg = (sg0, sg1, sg2, sg3)
        so = (so0, so1, so2, so3)

        pre_descs = (
            pltpu.make_async_copy(
                i0.at[pl.ds(wbase, per_w)], idx_v.at[pl.ds(0, per_w)], sg0),
            pltpu.make_async_copy(
                i1.at[pl.ds(wbase, per_w)], idx_v.at[pl.ds(per_w, per_w)],
                sg0),
            pltpu.make_async_copy(
                g0.at[pl.ds(wbase, per_w)], g_v.at[pl.ds(0, per_w)], sg0),
            pltpu.make_async_copy(
                g1.at[pl.ds(wbase, per_w)], g_v.at[pl.ds(per_w, per_w)], sg0),
        )
        for d in pre_descs:
            d.start()
        for d in pre_descs:
            d.wait()

        def gather_descs(cc, b):
            base = cc * G
            d0 = pltpu.make_async_copy(
                table.at[idx_v.at[pl.ds(base, G)]],
                r_v.at[b, pl.ds(0, G)], sg[b])
            d1 = pltpu.make_async_copy(
                table.at[idx_v.at[pl.ds(per_w + base, G)]],
                r_v.at[b, pl.ds(G, G)], sg[b])
            return (d0, d1)

        def out_desc(cc, b):
            return pltpu.make_async_copy(
                o_v.at[b], out.at[pl.ds(wbase + cc * G, G)], so[b])

        for pre in range(3):
            for d in gather_descs(pre, pre):
                d.start()

        @pl.loop(0, n_chunks, step=4)
        def _quad(c):
            for b in range(4):
                cc = c + b
                for d in gather_descs(cc, b):
                    d.wait()

                @pl.when(cc + 3 < n_chunks)
                def _():
                    for d in gather_descs(cc + 3, (b + 3) % 4):
                        d.start()

                @pl.when(cc >= 4)
                def _():
                    out_desc(cc, b).wait()

                @pl.loop(0, G)
                def _tok(t):
                    ia = jnp.full((16,), cc * G + t, jnp.int32)
                    ga = plsc.load_gather(g_v, [ia])
                    gb = plsc.load_gather(g_v, [ia + per_w])
                    for j in range(H // 16):
                        sl = pl.ds(j * 16, 16)
                        o_v[b, t, sl] = (r_v[b, t, sl] * ga
                                         + r_v[b, G + t, sl] * gb)

                out_desc(cc, b).start()

        for tail in range(4):
            out_desc(n_chunks - 4 + tail, tail).wait()

    return combine


def kernel(hidden_states, expert_outputs, W1, b1, W2, b2):
    B, S, H = hidden_states.shape
    E = W2.shape[0]
    N = B * S
    x = hidden_states.reshape(N, H)
    table = expert_outputs.reshape(E * N, H)
    i0r, i1r, g0r, g1r = _router(
        x, W1.T, b1.reshape(1, H), W2, b2.reshape(E, 1))
    i0 = i0r.reshape(N)
    i1 = i1r.reshape(N)
    g0 = g0r.reshape(N)
    g1 = g1r.reshape(N)
    out = _make_combine(N, H)(table, i0, i1, g0, g1)
    return out.reshape(B, S, H)


# speedup vs baseline: 1.4586x; 1.0573x over previous
import dataclasses
import functools

import jax
import jax.numpy as jnp
from jax import lax
from jax.experimental import pallas as pl
from jax.experimental.pallas import tpu as pltpu
from jax.experimental.pallas import tpu_sc as plsc


def _router_body(T, E, n_total, x_ref, w1_ref, b1_ref, w2_ref, b2_ref,
                 i0_ref, i1_ref, g0_ref, g1_ref):
    i = pl.program_id(0)
    h = lax.dot_general(x_ref[...], w1_ref[...], (((1,), (1,)), ((), ())),
                        preferred_element_type=jnp.float32)
    h = jnp.maximum(h + b1_ref[...], 0.0)
    logits = lax.dot_general(w2_ref[...], h, (((1,), (1,)), ((), ())),
                             preferred_element_type=jnp.float32)
    logits = logits + b2_ref[...]
    m = jnp.max(logits, axis=0, keepdims=True)
    p = jnp.exp(logits - m)
    p = p / jnp.sum(p, axis=0, keepdims=True)
    sub = lax.broadcasted_iota(jnp.int32, (E, T), 0)
    p1 = jnp.max(p, axis=0, keepdims=True)
    e1 = jnp.min(jnp.where(p == p1, sub, E), axis=0, keepdims=True)
    pm = jnp.where(sub == e1, -jnp.inf, p)
    p2 = jnp.max(pm, axis=0, keepdims=True)
    e2 = jnp.min(jnp.where(pm == p2, sub, E), axis=0, keepdims=True)
    tok = i * T + lax.broadcasted_iota(jnp.int32, (1, T), 1)
    i0_ref[0] = e1 * n_total + tok
    i1_ref[0] = e2 * n_total + tok
    g0_ref[0] = p1
    g1_ref[0] = p2


def _router(x, w1, b1, w2, b2, T=1024):
    N, H = x.shape
    E = w2.shape[0]
    nb = N // T
    body = functools.partial(_router_body, T, E, N)
    outs = pl.pallas_call(
        body,
        grid=(nb,),
        in_specs=[
            pl.BlockSpec((T, H), lambda i: (i, 0)),
            pl.BlockSpec((H, H), lambda i: (0, 0)),
            pl.BlockSpec((1, H), lambda i: (0, 0)),
            pl.BlockSpec((E, H), lambda i: (0, 0)),
            pl.BlockSpec((E, 1), lambda i: (0, 0)),
        ],
        out_specs=[
            pl.BlockSpec((1, 1, T), lambda i: (i, 0, 0)),
            pl.BlockSpec((1, 1, T), lambda i: (i, 0, 0)),
            pl.BlockSpec((1, 1, T), lambda i: (i, 0, 0)),
            pl.BlockSpec((1, 1, T), lambda i: (i, 0, 0)),
        ],
        out_shape=[
            jax.ShapeDtypeStruct((nb, 1, T), jnp.int32),
            jax.ShapeDtypeStruct((nb, 1, T), jnp.int32),
            jax.ShapeDtypeStruct((nb, 1, T), jnp.float32),
            jax.ShapeDtypeStruct((nb, 1, T), jnp.float32),
        ],
    )(x, w1, b1, w2, b2)
    return outs


def _make_combine(N, H, G=8):
    n_workers = 32
    per_w = N // n_workers
    n_chunks = per_w // G
    assert n_chunks % 4 == 0
    mesh = plsc.VectorSubcoreMesh(
        core_axis_name="c", subcore_axis_name="s", num_cores=2, num_subcores=16)

    cp = pltpu.CompilerParams()
    if "needs_layout_passes" in pltpu.CompilerParams.__dataclass_fields__:
        cp = dataclasses.replace(cp, needs_layout_passes=False)

    @functools.partial(
        pl.kernel,
        out_type=jax.ShapeDtypeStruct((N, H), jnp.float32),
        mesh=mesh,
        compiler_params=cp,
        scratch_types=[
            pltpu.VMEM((2 * per_w,), jnp.int32),
            pltpu.VMEM((2 * per_w,), jnp.float32),
            pltpu.VMEM((4, 2 * G, H), jnp.float32),
            pltpu.VMEM((4, G, H), jnp.float32),
            pltpu.SemaphoreType.DMA,
            pltpu.SemaphoreType.DMA,
            pltpu.SemaphoreType.DMA,
            pltpu.SemaphoreType.DMA,
            pltpu.SemaphoreType.DMA,
            pltpu.SemaphoreType.DMA,
            pltpu.SemaphoreType.DMA,
            pltpu.SemaphoreType.DMA,
        ],
    )
    def combine(table, i0, i1, g0, g1, out,
                idx_v, g_v, r_v, o_v,
                sg0, sg1, sg2, sg3, so0, so1, so2, so3):
        wid = lax.axis_index("s") * 2 + lax.axis_index("c")
        wbase = wid * per_w
        sg = (sg0, sg1, sg2, sg3)
        so = (so0, so1, so2, so3)

        pre_descs = (
            pltpu.make_async_copy(
                i0.at[pl.ds(wbase, per_w)], idx_v.at[pl.ds(0, per_w)], sg0),
            pltpu.make_async_copy(
                i1.at[pl.ds(wbase, per_w)], idx_v.at[pl.ds(per_w, per_w)],
                sg0),
            pltpu.make_async_copy(
                g0.at[pl.ds(wbase, per_w)], g_v.at[pl.ds(0, per_w)], sg0),
            pltpu.make_async_copy(
                g1.at[pl.ds(wbase, per_w)], g_v.at[pl.ds(per_w, per_w)], sg0),
        )
        for d in pre_descs:
            d.start()
        for d in pre_descs:
            d.wait()

        def gather_descs(cc, b):
            base = cc * G
            d0 = pltpu.make_async_copy(
                table.at[idx_v.at[pl.ds(base, G)]],
                r_v.at[b, pl.ds(0, G)], sg[b])
            d1 = pltpu.make_async_copy(
                table.at[idx_v.at[pl.ds(per_w + base, G)]],
                r_v.at[b, pl.ds(G, G)], sg[b])
            return (d0, d1)

        def out_desc(cc, b):
            return pltpu.make_async_copy(
                o_v.at[b], out.at[pl.ds(wbase + cc * G, G)], so[b])

        for pre in range(3):
            for d in gather_descs(pre, pre):
                d.start()

        @pl.loop(0, n_chunks, step=4)
        def _quad(c):
            for b in range(4):
                cc = c + b
                for d in gather_descs(cc, b):
                    d.wait()

                @pl.when(cc + 3 < n_chunks)
                def _():
                    for d in gather_descs(cc + 3, (b + 3) % 4):
                        d.start()

                @pl.when(cc >= 4)
                def _():
                    out_desc(cc, b).wait()

                @pl.loop(0, G)
                def _tok(t):
                    ia = jnp.full((16,), cc * G + t, jnp.int32)
                    ga = plsc.load_gather(g_v, [ia])
                    gb = plsc.load_gather(g_v, [ia + per_w])
                    for j in range(H // 16):
                        sl = pl.ds(j * 16, 16)
                        o_v[b, t, sl] = (r_v[b, t, sl] * ga
                                         + r_v[b, G + t, sl] * gb)

                out_desc(cc, b).start()

        for tail in range(4):
            out_desc(n_chunks - 4 + tail, tail).wait()

    return combine


def kernel(hidden_states, expert_outputs, W1, b1, W2, b2):
    B, S, H = hidden_states.shape
    E = W2.shape[0]
    N = B * S
    x = hidden_states.reshape(N, H)
    table = expert_outputs.reshape(E * N, H)
    i0r, i1r, g0r, g1r = _router(
        x, W1, b1.reshape(1, H), W2, b2.reshape(E, 1))
    i0 = i0r.reshape(N)
    i1 = i1r.reshape(N)
    g0 = g0r.reshape(N)
    g1 = g1r.reshape(N)
    out = _make_combine(N, H)(table, i0, i1, g0, g1)
    return out.reshape(B, S, H)
